# Initial kernel scaffold; baseline (speedup 1.0000x reference)
import jax
import jax.numpy as jnp
from jax.experimental import pallas as pl


def kernel(input_ids, emb_table, tool_semantics, profiles, W_pe, b_pe):
    raise NotImplementedError("write your pallas kernel here")


# speedup vs baseline: 1.8237x; 1.8237x over previous
import functools

import jax
import jax.numpy as jnp
from jax import lax
from jax.experimental import pallas as pl
from jax.experimental.pallas import tpu as pltpu
from jax.experimental.pallas import tpu_sc as plsc

HIDDEN = 1024
NUM_NEW = 512
START = 99488

_NC = 2
_NS = 16
_NW = _NC * _NS


def _encode_body(sem_ref, prof_ref, w_ref, b_ref, out_ref):
    out_ref[...] = (
        sem_ref[...]
        + jnp.dot(prof_ref[...], w_ref[...], preferred_element_type=jnp.float32)
        + b_ref[...]
    )


def _encode(tool_semantics, profiles, W_pe, b_pe):
    return pl.pallas_call(
        _encode_body,
        out_shape=jax.ShapeDtypeStruct((NUM_NEW, HIDDEN), jnp.float32),
    )(tool_semantics, profiles, W_pe, b_pe.reshape(1, HIDDEN))


def _make_sc_gather(n_tokens):
    tok_w = n_tokens // _NW
    C = 64
    F = 16
    n_chunks = tok_w // C
    n_groups = tok_w // 16
    trash = n_tokens
    n_out = n_tokens + 8
    mesh = plsc.VectorSubcoreMesh(
        core_axis_name="c", subcore_axis_name="s",
        num_cores=_NC, num_subcores=_NS)

    @functools.partial(
        pl.kernel,
        out_type=jax.ShapeDtypeStruct((n_out, HIDDEN), jnp.float32),
        mesh=mesh,
        scratch_types=[
            pltpu.VMEM((tok_w,), jnp.int32),
            pltpu.VMEM((C, HIDDEN), jnp.float32),
            pltpu.VMEM((F, HIDDEN), jnp.float32),
            pltpu.VMEM((tok_w + F,), jnp.int32),
            pltpu.VMEM((tok_w + F,), jnp.int32),
            pltpu.SemaphoreType.DMA,
            pltpu.SemaphoreType.DMA,
        ],
        compiler_params=pltpu.CompilerParams(needs_layout_passes=False),
    )
    def body(ids_hbm, emb_hbm, new_hbm, out_hbm,
             ids_v, rows_v, fix_v, pos_b, rel_b, gsem, fsem):
        wid = lax.axis_index("s") * _NC + lax.axis_index("c")
        base = wid * tok_w
        pltpu.sync_copy(ids_hbm.at[pl.ds(base, tok_w)], ids_v)

        def main_chunk(k, carry):
            idx = ids_v.at[pl.ds(k * C, C)]
            pltpu.async_copy(emb_hbm.at[idx], rows_v, gsem).wait()
            pltpu.sync_copy(rows_v, out_hbm.at[pl.ds(base + k * C, C)])
            return carry

        lax.fori_loop(0, n_chunks, main_chunk, 0)

        def compact(g, n):
            idv = ids_v[pl.ds(g * 16, 16)]
            m = idv >= START
            ones = jnp.where(m, jnp.int32(1), jnp.int32(0))
            prefix = plsc.cumsum(ones)
            tgt = jnp.maximum(n + prefix - 1, 0)
            posv = base + g * 16 + lax.iota(jnp.int32, 16)
            relv = idv - START
            plsc.store_scatter(pos_b, [tgt], posv, mask=m)
            plsc.store_scatter(rel_b, [tgt], relv, mask=m)
            return n + jnp.sum(ones)

        n_new = lax.fori_loop(0, n_groups, compact, jnp.int32(0))

        pos_b[pl.ds(n_new, F)] = jnp.full((F,), trash, jnp.int32)
        rel_b[pl.ds(n_new, F)] = jnp.zeros((F,), jnp.int32)

        n_fix = (n_new + F - 1) // F

        def fix_chunk(k, carry):
            relv = rel_b[pl.ds(k * F, F)]
            pltpu.async_copy(new_hbm.at[relv], fix_v, fsem).wait()
            posv = pos_b[pl.ds(k * F, F)]
            pltpu.async_copy(fix_v, out_hbm.at[posv], fsem).wait()
            return carry

        lax.fori_loop(0, n_fix, fix_chunk, 0)

    return body


def kernel(input_ids, emb_table, tool_semantics, profiles, W_pe, b_pe):
    b, s = input_ids.shape
    n = b * s
    new_embeds = _encode(tool_semantics, profiles, W_pe, b_pe)
    ids_flat = input_ids.reshape(n).astype(jnp.int32)
    out = _make_sc_gather(n)(ids_flat, emb_table, new_embeds)
    return out[:n].reshape(b, s, HIDDEN)


# speedup vs baseline: 1.8804x; 1.0311x over previous
import functools

import jax
import jax.numpy as jnp
from jax import lax
from jax.experimental import pallas as pl
from jax.experimental.pallas import tpu as pltpu
from jax.experimental.pallas import tpu_sc as plsc

HIDDEN = 1024
NUM_NEW = 512
START = 99488

_NC = 2
_NS = 16
_NW = _NC * _NS


def _encode_body(sem_ref, prof_ref, w_ref, b_ref, out_ref):
    out_ref[...] = (
        sem_ref[...]
        + jnp.dot(prof_ref[...], w_ref[...], preferred_element_type=jnp.float32)
        + b_ref[...]
    )


def _encode(tool_semantics, profiles, W_pe, b_pe):
    return pl.pallas_call(
        _encode_body,
        out_shape=jax.ShapeDtypeStruct((NUM_NEW, HIDDEN), jnp.float32),
    )(tool_semantics, profiles, W_pe, b_pe.reshape(1, HIDDEN))


def _make_sc_gather(n_tokens):
    tok_w = n_tokens // _NW
    C = 32
    NB = 2
    F = 16
    n_chunks = tok_w // C
    n_super = n_chunks // NB
    n_groups = tok_w // 16
    trash = n_tokens
    n_out = n_tokens + 8
    mesh = plsc.VectorSubcoreMesh(
        core_axis_name="c", subcore_axis_name="s",
        num_cores=_NC, num_subcores=_NS)

    @functools.partial(
        pl.kernel,
        out_type=jax.ShapeDtypeStruct((n_out, HIDDEN), jnp.float32),
        mesh=mesh,
        scratch_types=[
            pltpu.VMEM((tok_w,), jnp.int32),
            pltpu.VMEM((NB, C, HIDDEN), jnp.float32),
            pltpu.VMEM((F, HIDDEN), jnp.float32),
            pltpu.VMEM((tok_w + F,), jnp.int32),
            pltpu.VMEM((tok_w + F,), jnp.int32),
            [pltpu.SemaphoreType.DMA] * NB,
            [pltpu.SemaphoreType.DMA] * NB,
            pltpu.SemaphoreType.DMA,
        ],
        compiler_params=pltpu.CompilerParams(needs_layout_passes=False),
    )
    def body(ids_hbm, emb_hbm, new_hbm, out_hbm,
             ids_v, rows_v, fix_v, pos_b, rel_b, gsems, ssems, fsem):
        wid = lax.axis_index("s") * _NC + lax.axis_index("c")
        base = wid * tok_w
        pltpu.sync_copy(ids_hbm.at[pl.ds(base, tok_w)], ids_v)

        def super_iter(g, carry):
            for b in range(NB):
                k = g * NB + b

                @pl.when(g > 0)
                def _drain_prev_scatter(b=b):
                    pltpu.make_async_copy(
                        rows_v.at[b], out_hbm.at[pl.ds(base, C)], ssems[b]
                    ).wait()

                pltpu.async_copy(
                    emb_hbm.at[ids_v.at[pl.ds(k * C, C)]], rows_v.at[b], gsems[b]
                )
            for b in range(NB):
                k = g * NB + b
                pltpu.make_async_copy(
                    emb_hbm.at[ids_v.at[pl.ds(k * C, C)]], rows_v.at[b], gsems[b]
                ).wait()
                pltpu.async_copy(
                    rows_v.at[b], out_hbm.at[pl.ds(base + k * C, C)], ssems[b]
                )
            return carry

        lax.fori_loop(0, n_super, super_iter, 0)
        for b in range(NB):
            pltpu.make_async_copy(
                rows_v.at[b], out_hbm.at[pl.ds(base, C)], ssems[b]
            ).wait()

        def compact(g, n):
            idv = ids_v[pl.ds(g * 16, 16)]
            m = idv >= START
            ones = jnp.where(m, jnp.int32(1), jnp.int32(0))
            prefix = plsc.cumsum(ones)
            tgt = jnp.maximum(n + prefix - 1, 0)
            posv = base + g * 16 + lax.iota(jnp.int32, 16)
            relv = idv - START
            plsc.store_scatter(pos_b, [tgt], posv, mask=m)
            plsc.store_scatter(rel_b, [tgt], relv, mask=m)
            return n + jnp.sum(ones)

        n_new = lax.fori_loop(0, n_groups, compact, jnp.int32(0))

        pos_b[pl.ds(n_new, F)] = jnp.full((F,), trash, jnp.int32)
        rel_b[pl.ds(n_new, F)] = jnp.zeros((F,), jnp.int32)

        n_fix = (n_new + F - 1) // F

        def fix_chunk(k, carry):
            relv = rel_b[pl.ds(k * F, F)]
            pltpu.async_copy(new_hbm.at[relv], fix_v, fsem).wait()
            posv = pos_b[pl.ds(k * F, F)]
            pltpu.async_copy(fix_v, out_hbm.at[posv], fsem).wait()
            return carry

        lax.fori_loop(0, n_fix, fix_chunk, 0)

    return body


def kernel(input_ids, emb_table, tool_semantics, profiles, W_pe, b_pe):
    b, s = input_ids.shape
    n = b * s
    new_embeds = _encode(tool_semantics, profiles, W_pe, b_pe)
    ids_flat = input_ids.reshape(n).astype(jnp.int32)
    out = _make_sc_gather(n)(ids_flat, emb_table, new_embeds)
    return out[:n].reshape(b, s, HIDDEN)


# speedup vs baseline: 3.3495x; 1.7812x over previous
import functools

import jax
import jax.numpy as jnp
from jax import lax
from jax.experimental import pallas as pl
from jax.experimental.pallas import tpu as pltpu
from jax.experimental.pallas import tpu_sc as plsc

HIDDEN = 1024
NUM_NEW = 512
START = 99488

_NC = 2
_NS = 16
_NW = _NC * _NS


def _encode_body(sem_ref, prof_ref, w_ref, b_ref, out_ref):
    out_ref[...] = (
        sem_ref[...]
        + jnp.dot(prof_ref[...], w_ref[...], preferred_element_type=jnp.float32)
        + b_ref[...]
    )


def _encode(tool_semantics, profiles, W_pe, b_pe):
    return pl.pallas_call(
        _encode_body,
        out_shape=jax.ShapeDtypeStruct((NUM_NEW, HIDDEN), jnp.float32),
    )(tool_semantics, profiles, W_pe, b_pe.reshape(1, HIDDEN))


def _make_sc_gather(n_tokens):
    tok_w = n_tokens // _NW
    C = 32
    NB = 2
    F = 16
    n_chunks = tok_w // C
    n_super = n_chunks // NB
    n_groups = tok_w // 16
    mesh = plsc.VectorSubcoreMesh(
        core_axis_name="c", subcore_axis_name="s",
        num_cores=_NC, num_subcores=_NS)

    @functools.partial(
        pl.kernel,
        out_type=jax.ShapeDtypeStruct((n_tokens, HIDDEN), jnp.float32),
        mesh=mesh,
        scratch_types=[
            pltpu.VMEM((tok_w,), jnp.int32),
            pltpu.VMEM((NB, C, HIDDEN), jnp.float32),
            pltpu.VMEM((F, HIDDEN), jnp.float32),
            pltpu.VMEM((tok_w + F,), jnp.int32),
            [pltpu.SemaphoreType.DMA] * NB,
            [pltpu.SemaphoreType.DMA] * NB,
            pltpu.SemaphoreType.DMA,
        ],
        compiler_params=pltpu.CompilerParams(needs_layout_passes=False),
    )
    def body(ids_hbm, emb_hbm, new_hbm, out_hbm,
             ids_v, rows_v, fix_v, cmb_b, gsems, ssems, fsem):
        wid = lax.axis_index("s") * _NC + lax.axis_index("c")
        base = wid * tok_w
        pltpu.sync_copy(ids_hbm.at[pl.ds(base, tok_w)], ids_v)

        def super_iter(g, carry):
            for b in range(NB):
                k = g * NB + b

                @pl.when(g > 0)
                def _drain_prev_scatter(b=b):
                    pltpu.make_async_copy(
                        rows_v.at[b], out_hbm.at[pl.ds(base, C)], ssems[b]
                    ).wait()

                pltpu.async_copy(
                    emb_hbm.at[ids_v.at[pl.ds(k * C, C)]], rows_v.at[b], gsems[b]
                )
            for b in range(NB):
                k = g * NB + b
                pltpu.make_async_copy(
                    emb_hbm.at[ids_v.at[pl.ds(k * C, C)]], rows_v.at[b], gsems[b]
                ).wait()
                pltpu.async_copy(
                    rows_v.at[b], out_hbm.at[pl.ds(base + k * C, C)], ssems[b]
                )
            return carry

        lax.fori_loop(0, n_super, super_iter, 0)
        for b in range(NB):
            pltpu.make_async_copy(
                rows_v.at[b], out_hbm.at[pl.ds(base, C)], ssems[b]
            ).wait()

        def compact(g, carry):
            n, last = carry
            idv = ids_v[pl.ds(g * 16, 16)]
            m = idv >= START
            ones = jnp.where(m, jnp.int32(1), jnp.int32(0))
            prefix = plsc.cumsum(ones)
            tgt = jnp.maximum(n + prefix - 1, 0)
            posv = base + g * 16 + lax.iota(jnp.int32, 16)
            cmbv = posv * 1024 + (idv - START)
            plsc.store_scatter(cmb_b, [tgt], cmbv, mask=m)
            gmax = jnp.max(jnp.where(m, cmbv, jnp.int32(-1)))
            return n + jnp.sum(ones), jnp.maximum(last, gmax)

        n_new, last_cmb = lax.fori_loop(
            0, n_groups, compact, (jnp.int32(0), jnp.int32(0)))

        cmb_b[pl.ds(n_new, F)] = jnp.zeros((F,), jnp.int32) + last_cmb

        n_fix = (n_new + F - 1) // F

        def fix_chunk(k, carry):
            cmbv = cmb_b[pl.ds(k * F, F)]
            relv = jnp.bitwise_and(cmbv, jnp.int32(1023))
            posv = jnp.right_shift(cmbv, jnp.int32(10))
            pltpu.async_copy(new_hbm.at[relv], fix_v, fsem).wait()
            pltpu.async_copy(fix_v, out_hbm.at[posv], fsem).wait()
            return carry

        lax.fori_loop(0, n_fix, fix_chunk, 0)

    return body


def kernel(input_ids, emb_table, tool_semantics, profiles, W_pe, b_pe):
    b, s = input_ids.shape
    n = b * s
    new_embeds = _encode(tool_semantics, profiles, W_pe, b_pe)
    ids_flat = input_ids.reshape(n).astype(jnp.int32)
    out = _make_sc_gather(n)(ids_flat, emb_table, new_embeds)
    return out.reshape(b, s, HIDDEN)


# speedup vs baseline: 3.4441x; 1.0282x over previous
import functools

import jax
import jax.numpy as jnp
from jax import lax
from jax.experimental import pallas as pl
from jax.experimental.pallas import tpu as pltpu
from jax.experimental.pallas import tpu_sc as plsc

HIDDEN = 1024
NUM_NEW = 512
START = 99488

_NC = 2
_NS = 16
_NW = _NC * _NS


def _encode_body(sem_ref, prof_ref, w_ref, b_ref, out_ref):
    out_ref[...] = (
        sem_ref[...]
        + jnp.dot(prof_ref[...], w_ref[...], preferred_element_type=jnp.float32)
        + b_ref[...]
    )


def _encode(tool_semantics, profiles, W_pe, b_pe):
    return pl.pallas_call(
        _encode_body,
        out_shape=jax.ShapeDtypeStruct((NUM_NEW, HIDDEN), jnp.float32),
    )(tool_semantics, profiles, W_pe, b_pe.reshape(1, HIDDEN))


def _make_sc_gather(n_tokens):
    tok_w = n_tokens // _NW
    C = 16
    NB = 4
    F = 16
    n_chunks = tok_w // C
    n_super = n_chunks // NB
    n_groups = tok_w // 16
    mesh = plsc.VectorSubcoreMesh(
        core_axis_name="c", subcore_axis_name="s",
        num_cores=_NC, num_subcores=_NS)

    @functools.partial(
        pl.kernel,
        out_type=jax.ShapeDtypeStruct((n_tokens, HIDDEN), jnp.float32),
        mesh=mesh,
        scratch_types=[
            pltpu.VMEM((tok_w,), jnp.int32),
            pltpu.VMEM((NB, C, HIDDEN), jnp.float32),
            pltpu.VMEM((F, HIDDEN), jnp.float32),
            pltpu.VMEM((tok_w + F,), jnp.int32),
            [pltpu.SemaphoreType.DMA] * NB,
            [pltpu.SemaphoreType.DMA] * NB,
            pltpu.SemaphoreType.DMA,
        ],
        compiler_params=pltpu.CompilerParams(needs_layout_passes=False),
    )
    def body(ids_hbm, emb_hbm, new_hbm, out_hbm,
             ids_v, rows_v, fix_v, cmb_b, gsems, ssems, fsem):
        wid = lax.axis_index("s") * _NC + lax.axis_index("c")
        base = wid * tok_w
        pltpu.sync_copy(ids_hbm.at[pl.ds(base, tok_w)], ids_v)

        def super_iter(g, carry):
            for b in range(NB):
                k = g * NB + b

                @pl.when(g > 0)
                def _drain_prev_scatter(b=b):
                    pltpu.make_async_copy(
                        rows_v.at[b], out_hbm.at[pl.ds(base, C)], ssems[b]
                    ).wait()

                pltpu.async_copy(
                    emb_hbm.at[ids_v.at[pl.ds(k * C, C)]], rows_v.at[b], gsems[b]
                )
            for b in range(NB):
                k = g * NB + b
                pltpu.make_async_copy(
                    emb_hbm.at[ids_v.at[pl.ds(k * C, C)]], rows_v.at[b], gsems[b]
                ).wait()
                pltpu.async_copy(
                    rows_v.at[b], out_hbm.at[pl.ds(base + k * C, C)], ssems[b]
                )
            return carry

        lax.fori_loop(0, n_super, super_iter, 0)
        for b in range(NB):
            pltpu.make_async_copy(
                rows_v.at[b], out_hbm.at[pl.ds(base, C)], ssems[b]
            ).wait()

        def compact(g, carry):
            n, last = carry
            idv = ids_v[pl.ds(g * 16, 16)]
            m = idv >= START
            ones = jnp.where(m, jnp.int32(1), jnp.int32(0))
            prefix = plsc.cumsum(ones)
            tgt = jnp.maximum(n + prefix - 1, 0)
            posv = base + g * 16 + lax.iota(jnp.int32, 16)
            cmbv = posv * 1024 + (idv - START)
            plsc.store_scatter(cmb_b, [tgt], cmbv, mask=m)
            gmax = jnp.max(jnp.where(m, cmbv, jnp.int32(-1)))
            return n + jnp.sum(ones), jnp.maximum(last, gmax)

        n_new, last_cmb = lax.fori_loop(
            0, n_groups, compact, (jnp.int32(0), jnp.int32(0)))

        cmb_b[pl.ds(n_new, F)] = jnp.zeros((F,), jnp.int32) + last_cmb

        n_fix = (n_new + F - 1) // F

        def fix_chunk(k, carry):
            cmbv = cmb_b[pl.ds(k * F, F)]
            relv = jnp.bitwise_and(cmbv, jnp.int32(1023))
            posv = jnp.right_shift(cmbv, jnp.int32(10))
            pltpu.async_copy(new_hbm.at[relv], fix_v, fsem).wait()
            pltpu.async_copy(fix_v, out_hbm.at[posv], fsem).wait()
            return carry

        lax.fori_loop(0, n_fix, fix_chunk, 0)

    return body


def kernel(input_ids, emb_table, tool_semantics, profiles, W_pe, b_pe):
    b, s = input_ids.shape
    n = b * s
    new_embeds = _encode(tool_semantics, profiles, W_pe, b_pe)
    ids_flat = input_ids.reshape(n).astype(jnp.int32)
    out = _make_sc_gather(n)(ids_flat, emb_table, new_embeds)
    return out.reshape(b, s, HIDDEN)


# speedup vs baseline: 3.4622x; 1.0053x over previous
import functools

import jax
import jax.numpy as jnp
from jax import lax
from jax.experimental import pallas as pl
from jax.experimental.pallas import tpu as pltpu
from jax.experimental.pallas import tpu_sc as plsc

HIDDEN = 1024
NUM_NEW = 512
START = 99488

_NC = 2
_NS = 16
_NW = _NC * _NS


def _encode_body(sem_ref, prof_ref, w_ref, b_ref, out_ref):
    out_ref[...] = (
        sem_ref[...]
        + jnp.dot(prof_ref[...], w_ref[...], preferred_element_type=jnp.float32)
        + b_ref[...]
    )


def _encode(tool_semantics, profiles, W_pe, b_pe):
    return pl.pallas_call(
        _encode_body,
        out_shape=jax.ShapeDtypeStruct((NUM_NEW, HIDDEN), jnp.float32),
    )(tool_semantics, profiles, W_pe, b_pe.reshape(1, HIDDEN))


def _make_sc_gather(n_tokens):
    tok_w = n_tokens // _NW
    C = 8
    NB = 8
    F = 16
    n_chunks = tok_w // C
    n_super = n_chunks // NB
    n_groups = tok_w // 16
    mesh = plsc.VectorSubcoreMesh(
        core_axis_name="c", subcore_axis_name="s",
        num_cores=_NC, num_subcores=_NS)

    @functools.partial(
        pl.kernel,
        out_type=jax.ShapeDtypeStruct((n_tokens, HIDDEN), jnp.float32),
        mesh=mesh,
        scratch_types=[
            pltpu.VMEM((tok_w,), jnp.int32),
            pltpu.VMEM((NB, C, HIDDEN), jnp.float32),
            pltpu.VMEM((F, HIDDEN), jnp.float32),
            pltpu.VMEM((tok_w + F,), jnp.int32),
            [pltpu.SemaphoreType.DMA] * NB,
            [pltpu.SemaphoreType.DMA] * NB,
            pltpu.SemaphoreType.DMA,
        ],
        compiler_params=pltpu.CompilerParams(needs_layout_passes=False),
    )
    def body(ids_hbm, emb_hbm, new_hbm, out_hbm,
             ids_v, rows_v, fix_v, cmb_b, gsems, ssems, fsem):
        wid = lax.axis_index("s") * _NC + lax.axis_index("c")
        base = wid * tok_w
        pltpu.sync_copy(ids_hbm.at[pl.ds(base, tok_w)], ids_v)

        def super_iter(g, carry):
            for b in range(NB):
                k = g * NB + b

                @pl.when(g > 0)
                def _drain_prev_scatter(b=b):
                    pltpu.make_async_copy(
                        rows_v.at[b], out_hbm.at[pl.ds(base, C)], ssems[b]
                    ).wait()

                pltpu.async_copy(
                    emb_hbm.at[ids_v.at[pl.ds(k * C, C)]], rows_v.at[b], gsems[b]
                )
            for b in range(NB):
                k = g * NB + b
                pltpu.make_async_copy(
                    emb_hbm.at[ids_v.at[pl.ds(k * C, C)]], rows_v.at[b], gsems[b]
                ).wait()
                pltpu.async_copy(
                    rows_v.at[b], out_hbm.at[pl.ds(base + k * C, C)], ssems[b]
                )
            return carry

        lax.fori_loop(0, n_super, super_iter, 0)
        for b in range(NB):
            pltpu.make_async_copy(
                rows_v.at[b], out_hbm.at[pl.ds(base, C)], ssems[b]
            ).wait()

        def compact(g, carry):
            n, last = carry
            idv = ids_v[pl.ds(g * 16, 16)]
            m = idv >= START
            ones = jnp.where(m, jnp.int32(1), jnp.int32(0))
            prefix = plsc.cumsum(ones)
            tgt = jnp.maximum(n + prefix - 1, 0)
            posv = base + g * 16 + lax.iota(jnp.int32, 16)
            cmbv = posv * 1024 + (idv - START)
            plsc.store_scatter(cmb_b, [tgt], cmbv, mask=m)
            gmax = jnp.max(jnp.where(m, cmbv, jnp.int32(-1)))
            return n + jnp.sum(ones), jnp.maximum(last, gmax)

        n_new, last_cmb = lax.fori_loop(
            0, n_groups, compact, (jnp.int32(0), jnp.int32(0)))

        cmb_b[pl.ds(n_new, F)] = jnp.zeros((F,), jnp.int32) + last_cmb

        n_fix = (n_new + F - 1) // F

        def fix_chunk(k, carry):
            cmbv = cmb_b[pl.ds(k * F, F)]
            relv = jnp.bitwise_and(cmbv, jnp.int32(1023))
            posv = jnp.right_shift(cmbv, jnp.int32(10))
            pltpu.async_copy(new_hbm.at[relv], fix_v, fsem).wait()
            pltpu.async_copy(fix_v, out_hbm.at[posv], fsem).wait()
            return carry

        lax.fori_loop(0, n_fix, fix_chunk, 0)

    return body


def kernel(input_ids, emb_table, tool_semantics, profiles, W_pe, b_pe):
    b, s = input_ids.shape
    n = b * s
    new_embeds = _encode(tool_semantics, profiles, W_pe, b_pe)
    ids_flat = input_ids.reshape(n).astype(jnp.int32)
    out = _make_sc_gather(n)(ids_flat, emb_table, new_embeds)
    return out.reshape(b, s, HIDDEN)


# speedup vs baseline: 3.5379x; 1.0219x over previous
import functools

import jax
import jax.numpy as jnp
from jax import lax
from jax.experimental import pallas as pl
from jax.experimental.pallas import tpu as pltpu
from jax.experimental.pallas import tpu_sc as plsc

HIDDEN = 1024
NUM_NEW = 512
START = 99488

_NC = 2
_NS = 16
_NW = _NC * _NS


def _encode_body(sem_ref, prof_ref, w_ref, b_ref, out_ref):
    out_ref[...] = (
        sem_ref[...]
        + jnp.dot(prof_ref[...], w_ref[...], preferred_element_type=jnp.float32)
        + b_ref[...]
    )


def _encode(tool_semantics, profiles, W_pe, b_pe):
    return pl.pallas_call(
        _encode_body,
        out_shape=jax.ShapeDtypeStruct((NUM_NEW, HIDDEN), jnp.float32),
    )(tool_semantics, profiles, W_pe, b_pe.reshape(1, HIDDEN))


def _make_sc_gather(n_tokens):
    tok_w = n_tokens // _NW
    C = 8
    NB = 8
    F = 16
    n_chunks = tok_w // C
    n_super = n_chunks // NB
    n_groups = tok_w // 16
    mesh = plsc.VectorSubcoreMesh(
        core_axis_name="c", subcore_axis_name="s",
        num_cores=_NC, num_subcores=_NS)

    @functools.partial(
        pl.kernel,
        out_type=jax.ShapeDtypeStruct((n_tokens, HIDDEN), jnp.float32),
        mesh=mesh,
        scratch_types=[
            pltpu.VMEM((tok_w,), jnp.int32),
            pltpu.VMEM((NB, C, HIDDEN), jnp.float32),
            pltpu.VMEM((F, HIDDEN), jnp.float32),
            pltpu.VMEM((tok_w + F,), jnp.int32),
            [pltpu.SemaphoreType.DMA] * NB,
            [pltpu.SemaphoreType.DMA] * NB,
            pltpu.SemaphoreType.DMA,
        ],
        compiler_params=pltpu.CompilerParams(needs_layout_passes=False),
    )
    def body(ids_hbm, emb_hbm, new_hbm, out_hbm,
             ids_v, rows_v, fix_v, cmb_b, gsems, ssems, fsem):
        wid = lax.axis_index("s") * _NC + lax.axis_index("c")
        base = wid * tok_w
        w_per_row = ids_hbm.shape[1] // tok_w
        pltpu.sync_copy(
            ids_hbm.at[wid // w_per_row,
                       pl.ds((wid % w_per_row) * tok_w, tok_w)],
            ids_v)

        def super_iter(g, carry):
            for b in range(NB):
                k = g * NB + b

                @pl.when(g > 0)
                def _drain_prev_scatter(b=b):
                    pltpu.make_async_copy(
                        rows_v.at[b], out_hbm.at[pl.ds(base, C)], ssems[b]
                    ).wait()

                pltpu.async_copy(
                    emb_hbm.at[ids_v.at[pl.ds(k * C, C)]], rows_v.at[b], gsems[b]
                )
            for b in range(NB):
                k = g * NB + b
                pltpu.make_async_copy(
                    emb_hbm.at[ids_v.at[pl.ds(k * C, C)]], rows_v.at[b], gsems[b]
                ).wait()
                pltpu.async_copy(
                    rows_v.at[b], out_hbm.at[pl.ds(base + k * C, C)], ssems[b]
                )
            return carry

        lax.fori_loop(0, n_super, super_iter, 0)
        for b in range(NB):
            pltpu.make_async_copy(
                rows_v.at[b], out_hbm.at[pl.ds(base, C)], ssems[b]
            ).wait()

        def compact(g, carry):
            n, last = carry
            idv = ids_v[pl.ds(g * 16, 16)]
            m = idv >= START
            ones = jnp.where(m, jnp.int32(1), jnp.int32(0))
            prefix = plsc.cumsum(ones)
            tgt = jnp.maximum(n + prefix - 1, 0)
            posv = base + g * 16 + lax.iota(jnp.int32, 16)
            cmbv = posv * 1024 + (idv - START)
            plsc.store_scatter(cmb_b, [tgt], cmbv, mask=m)
            gmax = jnp.max(jnp.where(m, cmbv, jnp.int32(-1)))
            return n + jnp.sum(ones), jnp.maximum(last, gmax)

        n_new, last_cmb = lax.fori_loop(
            0, n_groups, compact, (jnp.int32(0), jnp.int32(0)))

        cmb_b[pl.ds(n_new, F)] = jnp.zeros((F,), jnp.int32) + last_cmb

        n_fix = (n_new + F - 1) // F

        def fix_chunk(k, carry):
            cmbv = cmb_b[pl.ds(k * F, F)]
            relv = jnp.bitwise_and(cmbv, jnp.int32(1023))
            posv = jnp.right_shift(cmbv, jnp.int32(10))
            pltpu.async_copy(new_hbm.at[relv], fix_v, fsem).wait()
            pltpu.async_copy(fix_v, out_hbm.at[posv], fsem).wait()
            return carry

        lax.fori_loop(0, n_fix, fix_chunk, 0)

    return body


def kernel(input_ids, emb_table, tool_semantics, profiles, W_pe, b_pe):
    b, s = input_ids.shape
    n = b * s
    new_embeds = _encode(tool_semantics, profiles, W_pe, b_pe)
    ids_2d = input_ids.astype(jnp.int32)
    out = _make_sc_gather(n)(ids_2d, emb_table, new_embeds)
    return out.reshape(b, s, HIDDEN)


# speedup vs baseline: 3.5431x; 1.0015x over previous
import functools

import jax
import jax.numpy as jnp
from jax import lax
from jax.experimental import pallas as pl
from jax.experimental.pallas import tpu as pltpu
from jax.experimental.pallas import tpu_sc as plsc

HIDDEN = 1024
NUM_NEW = 512
START = 99488

_NC = 2
_NS = 16
_NW = _NC * _NS


def _encode_body(sem_ref, prof_ref, w_ref, b_ref, out_ref):
    out_ref[...] = (
        sem_ref[...]
        + jnp.dot(prof_ref[...], w_ref[...], preferred_element_type=jnp.float32)
        + b_ref[...]
    )


def _encode(tool_semantics, profiles, W_pe, b_pe):
    return pl.pallas_call(
        _encode_body,
        out_shape=jax.ShapeDtypeStruct((NUM_NEW, HIDDEN), jnp.float32),
    )(tool_semantics, profiles, W_pe, b_pe.reshape(1, HIDDEN))


def _make_sc_gather(n_tokens):
    tok_w = n_tokens // _NW
    C = 8
    NB = 8
    F = 16
    n_chunks = tok_w // C
    n_super = n_chunks // NB
    n_groups = tok_w // 16
    mesh = plsc.VectorSubcoreMesh(
        core_axis_name="c", subcore_axis_name="s",
        num_cores=_NC, num_subcores=_NS)

    @functools.partial(
        pl.kernel,
        out_type=jax.ShapeDtypeStruct((n_tokens, HIDDEN), jnp.float32),
        mesh=mesh,
        scratch_types=[
            pltpu.VMEM((tok_w,), jnp.int32),
            pltpu.VMEM((NB, C, HIDDEN), jnp.float32),
            pltpu.VMEM((F, HIDDEN), jnp.float32),
            pltpu.VMEM((tok_w + F,), jnp.int32),
            [pltpu.SemaphoreType.DMA] * NB,
            [pltpu.SemaphoreType.DMA] * NB,
            pltpu.SemaphoreType.DMA,
        ],
        compiler_params=pltpu.CompilerParams(needs_layout_passes=False),
    )
    def body(ids_hbm, emb_hbm, new_hbm, out_hbm,
             ids_v, rows_v, fix_v, cmb_b, gsems, ssems, fsem):
        wid = lax.axis_index("s") * _NC + lax.axis_index("c")
        base = wid * tok_w
        w_per_row = ids_hbm.shape[1] // tok_w
        pltpu.sync_copy(
            ids_hbm.at[wid // w_per_row,
                       pl.ds((wid % w_per_row) * tok_w, tok_w)],
            ids_v)

        def super_iter(g, carry):
            for b in range(NB):
                k = g * NB + b

                @pl.when(g > 0)
                def _drain_prev_scatter(b=b):
                    pltpu.make_async_copy(
                        rows_v.at[b], out_hbm.at[pl.ds(base, C)], ssems[b]
                    ).wait()

                pltpu.async_copy(
                    emb_hbm.at[ids_v.at[pl.ds(k * C, C)]], rows_v.at[b], gsems[b]
                )
            for b in range(NB):
                k = g * NB + b
                pltpu.make_async_copy(
                    emb_hbm.at[ids_v.at[pl.ds(k * C, C)]], rows_v.at[b], gsems[b]
                ).wait()
                pltpu.async_copy(
                    rows_v.at[b], out_hbm.at[pl.ds(base + k * C, C)], ssems[b]
                )
            return carry

        lax.fori_loop(0, n_super, super_iter, 0)
        for b in range(NB):
            pltpu.make_async_copy(
                rows_v.at[b], out_hbm.at[pl.ds(base, C)], ssems[b]
            ).wait()

        def compact(g, carry):
            n, last = carry
            idv = ids_v[pl.ds(g * 16, 16)]
            m = idv >= START
            ones = jnp.where(m, jnp.int32(1), jnp.int32(0))
            prefix = plsc.cumsum(ones)
            tgt = jnp.maximum(n + prefix - 1, 0)
            posv = base + g * 16 + lax.iota(jnp.int32, 16)
            cmbv = posv * 1024 + (idv - START)
            plsc.store_scatter(cmb_b, [tgt], cmbv, mask=m)
            gmax = jnp.max(jnp.where(m, cmbv, jnp.int32(-1)))
            return n + jnp.sum(ones), jnp.maximum(last, gmax)

        n_new, last_cmb = lax.fori_loop(
            0, n_groups, compact, (jnp.int32(0), jnp.int32(0)))

        cmb_b[pl.ds(n_new, F)] = jnp.zeros((F,), jnp.int32) + last_cmb

        n_fix = (n_new + F - 1) // F

        def fix_chunk(k, carry):
            cmbv = cmb_b[pl.ds(k * F, F)]
            relv = jnp.bitwise_and(cmbv, jnp.int32(1023))
            posv = jnp.right_shift(cmbv, jnp.int32(10))
            pltpu.async_copy(new_hbm.at[relv], fix_v, fsem).wait()
            pltpu.async_copy(fix_v, out_hbm.at[posv], fsem).wait()
            return carry

        lax.fori_loop(0, n_fix, fix_chunk, 0)

    return body


def kernel(input_ids, emb_table, tool_semantics, profiles, W_pe, b_pe):
    b, s = input_ids.shape
    n = b * s
    new_embeds = _encode(tool_semantics, profiles, W_pe, b_pe)
    if input_ids.dtype != jnp.int32:
        input_ids = input_ids.astype(jnp.int32)
    out = _make_sc_gather(n)(input_ids, emb_table, new_embeds)
    return out.reshape(b, s, HIDDEN)


# speedup vs baseline: 3.5486x; 1.0015x over previous
import functools

import jax
import jax.numpy as jnp
from jax import lax
from jax.experimental import pallas as pl
from jax.experimental.pallas import tpu as pltpu
from jax.experimental.pallas import tpu_sc as plsc

HIDDEN = 1024
NUM_NEW = 512
START = 99488

_NC = 2
_NS = 16
_NW = _NC * _NS


def _encode_body(sem_ref, prof_ref, w_ref, b_ref, out_ref):
    out_ref[...] = (
        sem_ref[...]
        + jnp.dot(prof_ref[...], w_ref[...], preferred_element_type=jnp.float32)
        + b_ref[...]
    )


def _encode(tool_semantics, profiles, W_pe, b_pe):
    return pl.pallas_call(
        _encode_body,
        out_shape=jax.ShapeDtypeStruct((NUM_NEW, HIDDEN), jnp.float32),
    )(tool_semantics, profiles, W_pe, b_pe.reshape(1, HIDDEN))


def _make_sc_gather(n_tokens):
    tok_w = n_tokens // _NW
    C = 8
    NB = 8
    F = 16
    n_chunks = tok_w // C
    n_super = n_chunks // NB
    n_groups = tok_w // 16
    mesh = plsc.VectorSubcoreMesh(
        core_axis_name="c", subcore_axis_name="s",
        num_cores=_NC, num_subcores=_NS)

    @functools.partial(
        pl.kernel,
        out_type=jax.ShapeDtypeStruct((n_tokens, HIDDEN), jnp.float32),
        mesh=mesh,
        scratch_types=[
            pltpu.VMEM((tok_w,), jnp.int32),
            pltpu.VMEM((NB, C, HIDDEN), jnp.float32),
            pltpu.VMEM((F, HIDDEN), jnp.float32),
            pltpu.VMEM((tok_w + F,), jnp.int32),
            [pltpu.SemaphoreType.DMA] * NB,
            [pltpu.SemaphoreType.DMA] * NB,
            pltpu.SemaphoreType.DMA,
        ],
        compiler_params=pltpu.CompilerParams(needs_layout_passes=False),
    )
    def body(ids_hbm, emb_hbm, new_hbm, out_hbm,
             ids_v, rows_v, fix_v, cmb_b, gsems, ssems, fsem):
        wid = lax.axis_index("s") * _NC + lax.axis_index("c")
        base = wid * tok_w
        w_per_row = ids_hbm.shape[1] // tok_w
        pltpu.sync_copy(
            ids_hbm.at[wid // w_per_row,
                       pl.ds((wid % w_per_row) * tok_w, tok_w)],
            ids_v)

        for b in range(NB):
            pltpu.async_copy(
                emb_hbm.at[ids_v.at[pl.ds(b * C, C)]], rows_v.at[b], gsems[b]
            )

        def compact(g, carry):
            n, last = carry
            idv = ids_v[pl.ds(g * 16, 16)]
            m = idv >= START
            ones = jnp.where(m, jnp.int32(1), jnp.int32(0))
            prefix = plsc.cumsum(ones)
            tgt = jnp.maximum(n + prefix - 1, 0)
            posv = base + g * 16 + lax.iota(jnp.int32, 16)
            cmbv = posv * 1024 + (idv - START)
            plsc.store_scatter(cmb_b, [tgt], cmbv, mask=m)
            gmax = jnp.max(jnp.where(m, cmbv, jnp.int32(-1)))
            return n + jnp.sum(ones), jnp.maximum(last, gmax)

        n_new, last_cmb = lax.fori_loop(
            0, n_groups, compact, (jnp.int32(0), jnp.int32(0)))

        cmb_b[pl.ds(n_new, F)] = jnp.zeros((F,), jnp.int32) + last_cmb

        for b in range(NB):
            pltpu.make_async_copy(
                emb_hbm.at[ids_v.at[pl.ds(b * C, C)]], rows_v.at[b], gsems[b]
            ).wait()
            pltpu.async_copy(
                rows_v.at[b], out_hbm.at[pl.ds(base + b * C, C)], ssems[b]
            )

        def super_iter(g, carry):
            for b in range(NB):
                k = g * NB + b
                pltpu.make_async_copy(
                    rows_v.at[b], out_hbm.at[pl.ds(base, C)], ssems[b]
                ).wait()
                pltpu.async_copy(
                    emb_hbm.at[ids_v.at[pl.ds(k * C, C)]], rows_v.at[b], gsems[b]
                )
            for b in range(NB):
                k = g * NB + b
                pltpu.make_async_copy(
                    emb_hbm.at[ids_v.at[pl.ds(k * C, C)]], rows_v.at[b], gsems[b]
                ).wait()
                pltpu.async_copy(
                    rows_v.at[b], out_hbm.at[pl.ds(base + k * C, C)], ssems[b]
                )
            return carry

        lax.fori_loop(1, n_super, super_iter, 0)
        for b in range(NB):
            pltpu.make_async_copy(
                rows_v.at[b], out_hbm.at[pl.ds(base, C)], ssems[b]
            ).wait()

        n_fix = (n_new + F - 1) // F

        def fix_chunk(k, carry):
            cmbv = cmb_b[pl.ds(k * F, F)]
            relv = jnp.bitwise_and(cmbv, jnp.int32(1023))
            posv = jnp.right_shift(cmbv, jnp.int32(10))
            pltpu.async_copy(new_hbm.at[relv], fix_v, fsem).wait()
            pltpu.async_copy(fix_v, out_hbm.at[posv], fsem).wait()
            return carry

        lax.fori_loop(0, n_fix, fix_chunk, 0)

    return body


def kernel(input_ids, emb_table, tool_semantics, profiles, W_pe, b_pe):
    b, s = input_ids.shape
    n = b * s
    new_embeds = _encode(tool_semantics, profiles, W_pe, b_pe)
    if input_ids.dtype != jnp.int32:
        input_ids = input_ids.astype(jnp.int32)
    out = _make_sc_gather(n)(input_ids, emb_table, new_embeds)
    return out.reshape(b, s, HIDDEN)
